# baseline (speedup 1.0000x reference)
import jax
import jax.numpy as jnp
from jax.experimental import pallas as pl

N_NODES = 10000
N_LAYERS = 3


def _spmm(idx, val, x, n):
    gathered = val[:, None] * jnp.take(x, idx[1], axis=0)
    return jax.ops.segment_sum(gathered, idx[0], num_segments=n)


def _copy_body(x_ref, o_ref):
    o_ref[...] = x_ref[...]


def _passthrough(x):
    return pl.pallas_call(
        _copy_body,
        out_shape=jax.ShapeDtypeStruct(x.shape, x.dtype),
    )(x)


def kernel(source_user_embedding, source_item_embedding, target_user_embedding,
           target_item_embedding, adj_s_idx, adj_s_val, adj_t_idx, adj_t_val):
    s = jnp.concatenate([source_user_embedding, source_item_embedding], axis=0)
    t = jnp.concatenate([target_user_embedding, target_item_embedding], axis=0)
    for _ in range(N_LAYERS):
        s = _spmm(adj_s_idx, adj_s_val, s, N_NODES)
        t = _spmm(adj_t_idx, adj_t_val, t, N_NODES)
    return (_passthrough(s), _passthrough(t))


# speedup vs baseline: 1.2020x; 1.2020x over previous
import functools

import jax
import jax.numpy as jnp
from jax import lax
from jax.experimental import pallas as pl
from jax.experimental.pallas import tpu as pltpu
from jax.experimental.pallas import tpu_sc as plsc

N = 10000
NP = 10240
D = 256
E = 160000
HALF = 128
NS = 16
LANE = 16
RPT = NP // NS
BCAP = 11264
EC = 2000
G = 32
JG = HALF // LANE


def _body(rows_hbm, cols_hbm, vals_hbm, x_hbm, out_hbm, l1_hbm, l2_hbm,
          b_rows, b_cols, b_vals, st_r, st_c, st_v, acc, gb0, gb1,
          sem0, sem1):
    c = lax.axis_index("c")
    s = lax.axis_index("s")
    lo = s * RPT
    hi = lo + RPT
    col_off = c * NP

    def chunk_body(ci, ptr):
        base = ci * EC
        pltpu.sync_copy(rows_hbm.at[pl.ds(base, EC)], st_r)
        pltpu.sync_copy(cols_hbm.at[pl.ds(base, EC)], st_c)
        pltpu.sync_copy(vals_hbm.at[pl.ds(base, EC)], st_v)

        def grp(gi, p):
            r16 = st_r[pl.ds(gi * LANE, LANE)]
            m = (r16 >= lo) & (r16 < hi)
            mi = m.astype(jnp.int32)
            cs = plsc.cumsum(mi)
            pos = p + cs - mi
            plsc.store_scatter(b_rows, [pos], r16 - lo, mask=m)
            c16 = st_c[pl.ds(gi * LANE, LANE)]
            plsc.store_scatter(b_cols, [pos], c16 + col_off, mask=m)
            v16 = st_v[pl.ds(gi * LANE, LANE)]
            plsc.store_scatter(b_vals, [pos], v16, mask=m)
            return p + cs[LANE - 1]

        return lax.fori_loop(0, EC // LANE, grp, ptr)

    nedge = lax.fori_loop(0, E // EC, chunk_body, jnp.int32(0))

    for t in range(2 * G // LANE):
        off = nedge + t * LANE
        b_rows[pl.ds(off, LANE)] = jnp.zeros((LANE,), jnp.int32)
        b_vals[pl.ds(off, LANE)] = jnp.zeros((LANE,), jnp.float32)
        b_cols[pl.ds(off, LANE)] = jnp.zeros((LANE,), jnp.int32) + col_off

    nb2 = jnp.maximum(2 * ((nedge + 2 * G - 1) // (2 * G)), 2)

    def compute(k, gb):
        base = k * G
        for h in range(G // LANE):
            r16 = b_rows[pl.ds(base + h * LANE, LANE)]
            v16 = b_vals[pl.ds(base + h * LANE, LANE)]
            for e in range(LANE):
                r = r16[e]
                v = v16[e]
                for j in range(JG):
                    plsc.addupdate(acc.at[r, pl.ds(j * LANE, LANE)],
                                   v * gb[h * LANE + e, pl.ds(j * LANE, LANE)])

    for src, dst in ((x_hbm, l1_hbm), (l1_hbm, l2_hbm), (l2_hbm, out_hbm)):
        def issue(k, gb, sem, src=src):
            return pltpu.async_copy(src.at[b_cols.at[pl.ds(k * G, G)]],
                                    gb, sem)

        issue(0, gb0, sem0)

        def zrow(r, carry):
            for j in range(JG):
                acc[r, pl.ds(j * LANE, LANE)] = jnp.zeros((LANE,),
                                                          jnp.float32)
            return carry

        lax.fori_loop(0, RPT, zrow, 0)

        def pair(k):
            @pl.when(k + 1 < nb2)
            def _():
                issue(k + 1, gb1, sem1)
            pltpu.make_async_copy(src.at[pl.ds(0, G)], gb0, sem0).wait()
            compute(k, gb0)

            @pl.when(k + 2 < nb2)
            def _():
                issue(k + 2, gb0, sem0)
            pltpu.make_async_copy(src.at[pl.ds(0, G)], gb1, sem1).wait()
            compute(k + 1, gb1)

        pl.loop(0, nb2, step=2)(pair)

        pltpu.sync_copy(acc, dst.at[pl.ds(col_off + lo, RPT)])
        plsc.subcore_barrier()


def _sc_propagate(x2, rows, cols, vals):
    mesh = plsc.VectorSubcoreMesh(core_axis_name="c", subcore_axis_name="s")
    shp = jax.ShapeDtypeStruct((2 * NP, HALF), jnp.float32)
    out, _, _ = pl.kernel(
        _body,
        out_type=(shp, shp, shp),
        mesh=mesh,
        compiler_params=pltpu.CompilerParams(needs_layout_passes=False),
        scratch_types=(
            pltpu.VMEM((BCAP,), jnp.int32),
            pltpu.VMEM((BCAP,), jnp.int32),
            pltpu.VMEM((BCAP,), jnp.float32),
            pltpu.VMEM((EC,), jnp.int32),
            pltpu.VMEM((EC,), jnp.int32),
            pltpu.VMEM((EC,), jnp.float32),
            pltpu.VMEM((RPT, HALF), jnp.float32),
            pltpu.VMEM((G, HALF), jnp.float32),
            pltpu.VMEM((G, HALF), jnp.float32),
            pltpu.SemaphoreType.DMA,
            pltpu.SemaphoreType.DMA,
        ),
    )(rows, cols, vals, x2)
    return out


def kernel(source_user_embedding, source_item_embedding,
           target_user_embedding, target_item_embedding,
           adj_s_idx, adj_s_val, adj_t_idx, adj_t_val):
    xs = jnp.concatenate([source_user_embedding, source_item_embedding], axis=0)
    xt = jnp.concatenate([target_user_embedding, target_item_embedding], axis=0)
    pad = jnp.zeros((NP - N, HALF), jnp.float32)
    xs2 = jnp.concatenate([xs[:, :HALF], pad, xs[:, HALF:], pad], axis=0)
    xt2 = jnp.concatenate([xt[:, :HALF], pad, xt[:, HALF:], pad], axis=0)
    os2 = _sc_propagate(xs2, adj_s_idx[0], adj_s_idx[1], adj_s_val)
    ot2 = _sc_propagate(xt2, adj_t_idx[0], adj_t_idx[1], adj_t_val)
    return (jnp.concatenate([os2[:N], os2[NP:NP + N]], axis=1),
            jnp.concatenate([ot2[:N], ot2[NP:NP + N]], axis=1))
